# baseline (speedup 1.0000x reference)
import jax
import jax.numpy as jnp
import numpy as np
from jax import lax
from jax.experimental import pallas as pl
from jax.experimental.pallas import tpu as pltpu
from jax.experimental.pallas import tpu_sc as plsc

_NUM_WORDS = 100000
_NUM_BUCKETS = 1 << 20
_MASK = _NUM_BUCKETS - 1
_K = 3
_D = 64
_DP = 128
_B = 16384
_C = 5

_rs = np.random.RandomState(1139)
_HA = tuple(int(x) for x in _rs.randint(1, 21000, size=(_K,)))
_HB = tuple(int(x) for x in _rs.randint(0, _NUM_BUCKETS, size=(_K,)))

_NC, _NS, _L = 2, 16, 16
_NWK = _NC * _NS
_BPW = _B // _NWK
_CB = 32
_NCH = _BPW // _CB
_R = _C * _K

_HBLK = 128
_NHB = _NUM_BUCKETS // _HBLK
_HBW = _NHB // _NWK


def _hash(ids, k):
    return (((ids * _HA[k]) & _MASK) + _HB[k]) & _MASK


def _fmt_body(tT_hbm, tC_hbm, tab_hbm,
              inT0, inC0, inT1, inC1, ob0, ob1, lsem0, lsem1, wsem0, wsem1):
    wid = lax.axis_index("s") * _NC + lax.axis_index("c")
    base = wid * _HBW

    def fire(j, bT, bC, sem):
        h0 = (base + j) * _HBLK
        pltpu.async_copy(tT_hbm.at[:, pl.ds(h0, _HBLK)], bT, sem)
        pltpu.async_copy(tC_hbm.at[:, pl.ds(h0, _HBLK)], bC, sem)

    def wait_loads(bT, bC, sem):
        pltpu.make_async_copy(tT_hbm.at[:, pl.ds(0, _HBLK)], bT, sem).wait()
        pltpu.make_async_copy(tC_hbm.at[:, pl.ds(0, _HBLK)], bC, sem).wait()

    rowv = [(lax.iota(jnp.int32, _L) + l * _L) for l in range(_DP // _L)]

    def transpose_into(bT, bC, ob):
        for d in range(_D):
            dT = jnp.full((_L,), d, jnp.int32)
            dC = jnp.full((_L,), _D + d, jnp.int32)
            for l in range(_HBLK // _L):
                plsc.store_scatter(ob, [rowv[l], dT], bT[d, pl.ds(l * _L, _L)])
                plsc.store_scatter(ob, [rowv[l], dC], bC[d, pl.ds(l * _L, _L)])

    def writeout(j, ob, wsem):
        h0 = (base + j) * _HBLK
        pltpu.async_copy(ob, tab_hbm.at[pl.ds(h0, _HBLK), :], wsem)

    def wait_write(ob, wsem):
        pltpu.make_async_copy(ob, tab_hbm.at[pl.ds(0, _HBLK), :], wsem).wait()

    fire(0, inT0, inC0, lsem0)

    def blk(i, carry):
        j0 = 2 * i
        fire(jnp.minimum(j0 + 1, _HBW - 1), inT1, inC1, lsem1)
        wait_loads(inT0, inC0, lsem0)

        @pl.when(i > 0)
        def _():
            wait_write(ob0, wsem0)

        transpose_into(inT0, inC0, ob0)
        writeout(j0, ob0, wsem0)
        fire(jnp.minimum(j0 + 2, _HBW - 1), inT0, inC0, lsem0)
        wait_loads(inT1, inC1, lsem1)

        @pl.when(i > 0)
        def _():
            wait_write(ob1, wsem1)

        transpose_into(inT1, inC1, ob1)
        writeout(j0 + 1, ob1, wsem1)
        return carry

    lax.fori_loop(0, _HBW // 2, blk, 0)
    wait_loads(inT0, inC0, lsem0)
    wait_write(ob0, wsem0)
    wait_write(ob1, wsem1)


def _body(tgt_hbm, ctx_hbm, tab_hbm, imp_hbm, out_hbm,
          tid_v, cid_v, idxT, idxC, wTr, wCr_v, rowsT, rowsC, out_v, sem):
    wid = lax.axis_index("s") * _NC + lax.axis_index("c")

    def chunk(ch, carry):
        base = wid * _BPW + ch * _CB
        pltpu.sync_copy(tgt_hbm.at[pl.ds(base, _CB)], tid_v)
        pltpu.sync_copy(ctx_hbm.at[pl.ds(base * _C, _CB * _C)], cid_v)
        for i in range(_CB // _L):
            lanes = lax.iota(jnp.int32, _L) + i * _L
            ids = tid_v[pl.ds(i * _L, _L)]
            for k in range(_K):
                idxT[k, pl.ds(i * _L, _L)] = _hash(ids, k)
            for c in range(_C):
                cids = plsc.load_gather(cid_v, [lanes * _C + c])
                idxC[_R + c, pl.ds(i * _L, _L)] = cids
                for k in range(_K):
                    idxC[c * _K + k, pl.ds(i * _L, _L)] = _hash(cids, k)
        cps = [pltpu.async_copy(imp_hbm.at[tid_v], wTr, sem)]
        for k in range(_K):
            cps.append(pltpu.async_copy(tab_hbm.at[idxT.at[k]], rowsT.at[k], sem))
        for r in range(_R):
            cps.append(pltpu.async_copy(tab_hbm.at[idxC.at[r]], rowsC.at[r], sem))
        for c in range(_C):
            cps.append(pltpu.async_copy(imp_hbm.at[idxC.at[_R + c]], wCr_v.at[c], sem))
        for cp in cps:
            cp.wait()
        for g in range(_CB // _L):
            b0 = g * _L
            blane = lax.iota(jnp.int32, _L) + b0
            ksp = [jnp.full((_L,), k, jnp.int32) for k in range(_K)]
            kcs = [jnp.full((_L,), _K + k, jnp.int32) for k in range(_K)]
            rsp = [jnp.full((_L,), r, jnp.int32) for r in range(_R)]
            csp = [jnp.full((_L,), c, jnp.int32) for c in range(_C)]
            wTk = [plsc.load_gather(wTr, [blane, ksp[k]]) for k in range(_K)]
            wCk = [plsc.load_gather(wCr_v, [csp[r // _K], blane, kcs[r % _K]])
                   for r in range(_R)]

            def dbody(d, accs):
                dsp = jnp.full((_L,), d, jnp.int32)
                dspc = jnp.full((_L,), d + _D, jnp.int32)
                we = wTk[0] * plsc.load_gather(rowsT, [ksp[0], blane, dsp])
                for k in range(1, _K):
                    we = we + wTk[k] * plsc.load_gather(rowsT, [ksp[k], blane, dsp])
                out = []
                for c in range(_C):
                    r0 = c * _K
                    ce = wCk[r0] * plsc.load_gather(rowsC, [rsp[r0], blane, dspc])
                    for k in range(1, _K):
                        r = r0 + k
                        ce = ce + wCk[r] * plsc.load_gather(rowsC, [rsp[r], blane, dspc])
                    out.append(accs[c] + we * ce)
                return tuple(out)

            accs = lax.fori_loop(
                0, _D, dbody,
                tuple(jnp.zeros((_L,), jnp.float32) for _ in range(_C)),
                unroll=4)
            for c in range(_C):
                plsc.store_scatter(out_v, [blane * _C + c], accs[c])
        pltpu.sync_copy(out_v, out_hbm.at[pl.ds(base * _C, _CB * _C)])
        return carry

    lax.fori_loop(0, _NCH, chunk, 0)


def kernel(target, context, table_target, imp_target, table_context, imp_context):
    tgt = target.reshape(_B).astype(jnp.int32)
    ctx = context.reshape(_B * _C).astype(jnp.int32)
    imp = jnp.pad(jnp.concatenate([imp_target, imp_context], axis=1),
                  ((0, 0), (0, _DP - 2 * _K)))
    mesh = plsc.VectorSubcoreMesh(core_axis_name="c", subcore_axis_name="s",
                                  num_cores=_NC, num_subcores=_NS)
    cparams = pltpu.CompilerParams(needs_layout_passes=False,
                                   use_tc_tiling_on_sc=True)

    tab = pl.kernel(
        _fmt_body,
        out_type=jax.ShapeDtypeStruct((_NUM_BUCKETS, _DP), jnp.float32),
        mesh=mesh,
        compiler_params=cparams,
        scratch_types=[
            pltpu.VMEM((_D, _HBLK), jnp.float32),
            pltpu.VMEM((_D, _HBLK), jnp.float32),
            pltpu.VMEM((_D, _HBLK), jnp.float32),
            pltpu.VMEM((_D, _HBLK), jnp.float32),
            pltpu.VMEM((_HBLK, _DP), jnp.float32),
            pltpu.VMEM((_HBLK, _DP), jnp.float32),
            pltpu.SemaphoreType.DMA,
            pltpu.SemaphoreType.DMA,
            pltpu.SemaphoreType.DMA,
            pltpu.SemaphoreType.DMA,
        ],
    )(table_target.T, table_context.T)

    out = pl.kernel(
        _body,
        out_type=jax.ShapeDtypeStruct((_B * _C,), jnp.float32),
        mesh=mesh,
        compiler_params=cparams,
        scratch_types=[
            pltpu.VMEM((_CB,), jnp.int32),
            pltpu.VMEM((_CB * _C,), jnp.int32),
            pltpu.VMEM((_K, _CB), jnp.int32),
            pltpu.VMEM((_R + _C, _CB), jnp.int32),
            pltpu.VMEM((_CB, _DP), jnp.float32),
            pltpu.VMEM((_C, _CB, _DP), jnp.float32),
            pltpu.VMEM((_K, _CB, _DP), jnp.float32),
            pltpu.VMEM((_R, _CB, _DP), jnp.float32),
            pltpu.VMEM((_CB * _C,), jnp.float32),
            pltpu.SemaphoreType.DMA,
        ],
    )(tgt, ctx, tab, imp)
    return out.reshape(_B, _C)


# speedup vs baseline: 1.0190x; 1.0190x over previous
import jax
import jax.numpy as jnp
import numpy as np
from jax import lax
from jax.experimental import pallas as pl
from jax.experimental.pallas import tpu as pltpu
from jax.experimental.pallas import tpu_sc as plsc

_NUM_WORDS = 100000
_NUM_BUCKETS = 1 << 20
_MASK = _NUM_BUCKETS - 1
_K = 3
_D = 64
_DP = 128
_B = 16384
_C = 5

_rs = np.random.RandomState(1139)
_HA = tuple(int(x) for x in _rs.randint(1, 21000, size=(_K,)))
_HB = tuple(int(x) for x in _rs.randint(0, _NUM_BUCKETS, size=(_K,)))

_NC, _NS, _L = 2, 16, 16
_NWK = _NC * _NS
_BPW = _B // _NWK
_CB = 32
_NCH = _BPW // _CB
_R = _C * _K

_HBLK = 128
_NHB = _NUM_BUCKETS // _HBLK
_HBW = _NHB // _NWK


def _hash(ids, k):
    return (((ids * _HA[k]) & _MASK) + _HB[k]) & _MASK


def _fmt_body(tT_hbm, tC_hbm, tab_hbm,
              inT0, inC0, inT1, inC1, ob0, ob1, lsem0, lsem1, wsem0, wsem1):
    wid = lax.axis_index("s") * _NC + lax.axis_index("c")
    base = wid * _HBW

    def fire(j, bT, bC, sem):
        h0 = (base + j) * _HBLK
        pltpu.async_copy(tT_hbm.at[:, pl.ds(h0, _HBLK)], bT, sem)
        pltpu.async_copy(tC_hbm.at[:, pl.ds(h0, _HBLK)], bC, sem)

    def wait_loads(bT, bC, sem):
        pltpu.make_async_copy(tT_hbm.at[:, pl.ds(0, _HBLK)], bT, sem).wait()
        pltpu.make_async_copy(tC_hbm.at[:, pl.ds(0, _HBLK)], bC, sem).wait()

    rowv = [(lax.iota(jnp.int32, _L) + l * _L) for l in range(_DP // _L)]

    def transpose_into(bT, bC, ob):
        def dstep(d, carry):
            dT = jnp.full((_L,), d, jnp.int32)
            dC = dT + _D
            for l in range(_HBLK // _L):
                plsc.store_scatter(ob, [rowv[l], dT], bT[d, pl.ds(l * _L, _L)])
                plsc.store_scatter(ob, [rowv[l], dC], bC[d, pl.ds(l * _L, _L)])
            return carry

        lax.fori_loop(0, _D, dstep, 0, unroll=2)

    def writeout(j, ob, wsem):
        h0 = (base + j) * _HBLK
        pltpu.async_copy(ob, tab_hbm.at[pl.ds(h0, _HBLK), :], wsem)

    def wait_write(ob, wsem):
        pltpu.make_async_copy(ob, tab_hbm.at[pl.ds(0, _HBLK), :], wsem).wait()

    fire(0, inT0, inC0, lsem0)

    def blk(i, carry):
        j0 = 2 * i
        fire(jnp.minimum(j0 + 1, _HBW - 1), inT1, inC1, lsem1)
        wait_loads(inT0, inC0, lsem0)

        @pl.when(i > 0)
        def _():
            wait_write(ob0, wsem0)

        transpose_into(inT0, inC0, ob0)
        writeout(j0, ob0, wsem0)
        fire(jnp.minimum(j0 + 2, _HBW - 1), inT0, inC0, lsem0)
        wait_loads(inT1, inC1, lsem1)

        @pl.when(i > 0)
        def _():
            wait_write(ob1, wsem1)

        transpose_into(inT1, inC1, ob1)
        writeout(j0 + 1, ob1, wsem1)
        return carry

    lax.fori_loop(0, _HBW // 2, blk, 0)
    wait_loads(inT0, inC0, lsem0)
    wait_write(ob0, wsem0)
    wait_write(ob1, wsem1)


def _body(tgt_hbm, ctx_hbm, tab_hbm, imp_hbm, out_hbm,
          tid_v, cid_v, idxT, idxC, wTr, wCr_v, rowsT, rowsC, out_v, sem):
    wid = lax.axis_index("s") * _NC + lax.axis_index("c")

    def chunk(ch, carry):
        base = wid * _BPW + ch * _CB
        pltpu.sync_copy(tgt_hbm.at[pl.ds(base, _CB)], tid_v)
        pltpu.sync_copy(ctx_hbm.at[pl.ds(base * _C, _CB * _C)], cid_v)
        for i in range(_CB // _L):
            lanes = lax.iota(jnp.int32, _L) + i * _L
            ids = tid_v[pl.ds(i * _L, _L)]
            for k in range(_K):
                idxT[k, pl.ds(i * _L, _L)] = _hash(ids, k)
            for c in range(_C):
                cids = plsc.load_gather(cid_v, [lanes * _C + c])
                idxC[_R + c, pl.ds(i * _L, _L)] = cids
                for k in range(_K):
                    idxC[c * _K + k, pl.ds(i * _L, _L)] = _hash(cids, k)
        cps = [pltpu.async_copy(imp_hbm.at[tid_v], wTr, sem)]
        for k in range(_K):
            cps.append(pltpu.async_copy(tab_hbm.at[idxT.at[k]], rowsT.at[k], sem))
        for r in range(_R):
            cps.append(pltpu.async_copy(tab_hbm.at[idxC.at[r]], rowsC.at[r], sem))
        for c in range(_C):
            cps.append(pltpu.async_copy(imp_hbm.at[idxC.at[_R + c]], wCr_v.at[c], sem))
        for cp in cps:
            cp.wait()
        for g in range(_CB // _L):
            b0 = g * _L
            blane = lax.iota(jnp.int32, _L) + b0
            ksp = [jnp.full((_L,), k, jnp.int32) for k in range(_K)]
            kcs = [jnp.full((_L,), _K + k, jnp.int32) for k in range(_K)]
            rsp = [jnp.full((_L,), r, jnp.int32) for r in range(_R)]
            csp = [jnp.full((_L,), c, jnp.int32) for c in range(_C)]
            wTk = [plsc.load_gather(wTr, [blane, ksp[k]]) for k in range(_K)]
            wCk = [plsc.load_gather(wCr_v, [csp[r // _K], blane, kcs[r % _K]])
                   for r in range(_R)]

            def dbody(d, accs):
                dsp = jnp.full((_L,), d, jnp.int32)
                dspc = jnp.full((_L,), d + _D, jnp.int32)
                we = wTk[0] * plsc.load_gather(rowsT, [ksp[0], blane, dsp])
                for k in range(1, _K):
                    we = we + wTk[k] * plsc.load_gather(rowsT, [ksp[k], blane, dsp])
                out = []
                for c in range(_C):
                    r0 = c * _K
                    ce = wCk[r0] * plsc.load_gather(rowsC, [rsp[r0], blane, dspc])
                    for k in range(1, _K):
                        r = r0 + k
                        ce = ce + wCk[r] * plsc.load_gather(rowsC, [rsp[r], blane, dspc])
                    out.append(accs[c] + we * ce)
                return tuple(out)

            accs = lax.fori_loop(
                0, _D, dbody,
                tuple(jnp.zeros((_L,), jnp.float32) for _ in range(_C)),
                unroll=4)
            for c in range(_C):
                plsc.store_scatter(out_v, [blane * _C + c], accs[c])
        pltpu.sync_copy(out_v, out_hbm.at[pl.ds(base * _C, _CB * _C)])
        return carry

    lax.fori_loop(0, _NCH, chunk, 0)


def kernel(target, context, table_target, imp_target, table_context, imp_context):
    tgt = target.reshape(_B).astype(jnp.int32)
    ctx = context.reshape(_B * _C).astype(jnp.int32)
    imp = jnp.pad(jnp.concatenate([imp_target, imp_context], axis=1),
                  ((0, 0), (0, _DP - 2 * _K)))
    mesh = plsc.VectorSubcoreMesh(core_axis_name="c", subcore_axis_name="s",
                                  num_cores=_NC, num_subcores=_NS)
    cparams = pltpu.CompilerParams(needs_layout_passes=False,
                                   use_tc_tiling_on_sc=True)

    tab = pl.kernel(
        _fmt_body,
        out_type=jax.ShapeDtypeStruct((_NUM_BUCKETS, _DP), jnp.float32),
        mesh=mesh,
        compiler_params=cparams,
        scratch_types=[
            pltpu.VMEM((_D, _HBLK), jnp.float32),
            pltpu.VMEM((_D, _HBLK), jnp.float32),
            pltpu.VMEM((_D, _HBLK), jnp.float32),
            pltpu.VMEM((_D, _HBLK), jnp.float32),
            pltpu.VMEM((_HBLK, _DP), jnp.float32),
            pltpu.VMEM((_HBLK, _DP), jnp.float32),
            pltpu.SemaphoreType.DMA,
            pltpu.SemaphoreType.DMA,
            pltpu.SemaphoreType.DMA,
            pltpu.SemaphoreType.DMA,
        ],
    )(table_target.T, table_context.T)

    out = pl.kernel(
        _body,
        out_type=jax.ShapeDtypeStruct((_B * _C,), jnp.float32),
        mesh=mesh,
        compiler_params=cparams,
        scratch_types=[
            pltpu.VMEM((_CB,), jnp.int32),
            pltpu.VMEM((_CB * _C,), jnp.int32),
            pltpu.VMEM((_K, _CB), jnp.int32),
            pltpu.VMEM((_R + _C, _CB), jnp.int32),
            pltpu.VMEM((_CB, _DP), jnp.float32),
            pltpu.VMEM((_C, _CB, _DP), jnp.float32),
            pltpu.VMEM((_K, _CB, _DP), jnp.float32),
            pltpu.VMEM((_R, _CB, _DP), jnp.float32),
            pltpu.VMEM((_CB * _C,), jnp.float32),
            pltpu.SemaphoreType.DMA,
        ],
    )(tgt, ctx, tab, imp)
    return out.reshape(_B, _C)
